# baseline (speedup 1.0000x reference)
import functools

import jax
import jax.numpy as jnp
from jax import lax
from jax.experimental import pallas as pl
from jax.experimental.pallas import tpu as pltpu
from jax.experimental.pallas import tpu_sc as plsc

_NB = 100
_NC = 2
_NS = 16
_TCH = 16


def kernel(X, y_values, x_min, x_max):
    N, C = X.shape
    NW = _NC * _NS
    NT = N // 128
    tiles_w = NT // (NW // 2)
    n_chunks = tiles_w // _TCH

    dx = (x_max - x_min) / jnp.float32(_NB)
    a = (1.0 / dx).astype(jnp.float32)
    b = (-x_min / dx).astype(jnp.float32)
    ab = jnp.concatenate(
        [jnp.tile(a[:, None], (1, 16)), jnp.tile(b[:, None], (1, 16))],
        axis=0).reshape(-1)
    _ST = 104
    ylo_u = lax.bitcast_convert_type(
        y_values[:, :_NB].astype(jnp.bfloat16), jnp.uint16).astype(jnp.uint32)
    dy_u = lax.bitcast_convert_type(
        (y_values[:, 1:] - y_values[:, :-1]).astype(jnp.bfloat16),
        jnp.uint16).astype(jnp.uint32)
    packed = ((ylo_u << 16) | dy_u).astype(jnp.int32)
    one_pad = jnp.int32(0x3F800000 & -65536)
    pF = jnp.full((C, _ST), one_pad, jnp.int32).at[:, :_NB].set(
        packed).reshape(-1)

    x4 = jnp.transpose(X).reshape(2, 8, NT, 128).transpose(0, 2, 1, 3)

    mesh = plsc.VectorSubcoreMesh(
        core_axis_name="c", subcore_axis_name="s",
        num_cores=_NC, num_subcores=_NS,
    )

    @functools.partial(
        pl.kernel,
        out_type=jax.ShapeDtypeStruct((2, NT, 8, 128), jnp.float32),
        mesh=mesh,
        compiler_params=pltpu.CompilerParams(needs_layout_passes=False),
        scratch_types=[
            pltpu.VMEM((2, _TCH, 8, 128), jnp.float32),
            pltpu.VMEM((2, _TCH, 8, 128), jnp.float32),
            pltpu.VMEM((C * 104,), jnp.int32),
            pltpu.VMEM((2 * C * 16,), jnp.float32),
            pltpu.SemaphoreType.DMA,
            pltpu.SemaphoreType.DMA,
            pltpu.SemaphoreType.DMA,
            pltpu.SemaphoreType.DMA,
        ],
    )
    def _run(x_hbm, pF_hbm, ab_hbm, out_hbm,
             xbuf, obuf, ptab, abv,
             sem_in0, sem_in1, sem_out0, sem_out1):
        wid = lax.axis_index("s") * _NC + lax.axis_index("c")
        grp = wid & 1
        base = (wid >> 1) * tiles_w

        pltpu.sync_copy(pF_hbm, ptab)
        pltpu.sync_copy(ab_hbm, abv)

        avs = [abv[pl.ds((grp * 8 + i) * 16, 16)] for i in range(8)]
        bvs = [abv[pl.ds((C + grp * 8 + i) * 16, 16)] for i in range(8)]
        col0 = grp * 8
        sems_in = (sem_in0, sem_in1)
        sems_out = (sem_out0, sem_out1)

        def start_in(slot, k):
            t0 = base + k * _TCH
            pltpu.async_copy(x_hbm.at[grp, pl.ds(t0, _TCH)],
                             xbuf.at[slot], sems_in[slot])

        def wait_in(slot):
            pltpu.make_async_copy(x_hbm.at[0, pl.ds(0, _TCH)],
                                  xbuf.at[slot], sems_in[slot]).wait()

        def start_out(slot, k):
            t0 = base + k * _TCH
            pltpu.async_copy(obuf.at[slot],
                             out_hbm.at[grp, pl.ds(t0, _TCH)],
                             sems_out[slot])

        def wait_out(slot):
            pltpu.make_async_copy(obuf.at[slot],
                                  out_hbm.at[0, pl.ds(0, _TCH)],
                                  sems_out[slot]).wait()

        def compute_chunk(slot):
            xv = xbuf.at[slot].reshape(_TCH * 8, 128)
            ov = obuf.at[slot].reshape(_TCH * 8, 128)
            himask = jnp.int32(-65536)
            for i in range(8):
                ptab_i = ptab.at[pl.ds((col0 + i) * _ST, _ST)]

                for half in (0, 4):

                    @plsc.parallel_loop(i, _TCH * 8, 8, unroll=2)
                    def _(r):
                        for jj in range(half, half + 4):
                            x = xv[r, pl.ds(16 * jj, 16)]
                            tt = x * avs[i] + bvs[i]
                            idx = tt.astype(jnp.int32)
                            frac = tt - idx.astype(jnp.float32)
                            w = plsc.load_gather(ptab_i, [idx])
                            ylo = plsc.bitcast(w & himask, jnp.float32)
                            dy = plsc.bitcast(w << 16, jnp.float32)
                            ov[r, pl.ds(16 * jj, 16)] = ylo + frac * dy

        start_in(0, 0)
        start_in(1, 1)

        def pair_step(p, _):
            k0 = 2 * p
            for sub in (0, 1):
                k = k0 + sub
                wait_in(sub)

                @pl.when(k >= 2)
                def _():
                    wait_out(sub)

                compute_chunk(sub)
                start_out(sub, k)

                @pl.when(k + 2 < n_chunks)
                def _():
                    start_in(sub, k + 2)
            return 0

        lax.fori_loop(0, n_chunks // 2, pair_step, 0)
        wait_out(0)
        wait_out(1)

    o4 = _run(x4, pF, ab)
    return jnp.transpose(o4.transpose(0, 2, 1, 3).reshape(C, N))


# speedup vs baseline: 1.3027x; 1.3027x over previous
import functools

import jax
import jax.numpy as jnp
from jax import lax
from jax.experimental import pallas as pl
from jax.experimental.pallas import tpu as pltpu
from jax.experimental.pallas import tpu_sc as plsc

_NB = 100
_NC = 2
_NS = 16
_TCH = 16


def kernel(X, y_values, x_min, x_max):
    N, C = X.shape
    NW = _NC * _NS
    NT = N // 128
    tiles_w = NT // (NW // 2)
    n_chunks = tiles_w // _TCH

    dx = (x_max - x_min) / jnp.float32(_NB)
    a = (1.0 / dx).astype(jnp.float32)
    b = (-x_min / dx).astype(jnp.float32)
    ab = jnp.concatenate(
        [jnp.tile(a[:, None], (1, 16)), jnp.tile(b[:, None], (1, 16))],
        axis=0).reshape(-1)
    _ST = 104
    ylo_u = lax.bitcast_convert_type(
        y_values[:, :_NB].astype(jnp.bfloat16), jnp.uint16).astype(jnp.uint32)
    dy_u = lax.bitcast_convert_type(
        (y_values[:, 1:] - y_values[:, :-1]).astype(jnp.bfloat16),
        jnp.uint16).astype(jnp.uint32)
    packed = ((ylo_u << 16) | dy_u).astype(jnp.int32)
    one_pad = jnp.int32(0x3F800000 & -65536)
    pF = jnp.full((C, _ST), one_pad, jnp.int32).at[:, :_NB].set(
        packed).reshape(-1)

    x4 = jnp.transpose(X).reshape(2, 8, NT, 128).transpose(0, 2, 1, 3)

    mesh = plsc.VectorSubcoreMesh(
        core_axis_name="c", subcore_axis_name="s",
        num_cores=_NC, num_subcores=_NS,
    )

    @functools.partial(
        pl.kernel,
        out_type=jax.ShapeDtypeStruct((2, NT, 8, 128), jnp.float32),
        mesh=mesh,
        compiler_params=pltpu.CompilerParams(needs_layout_passes=False),
        scratch_types=[
            pltpu.VMEM((2, _TCH, 8, 128), jnp.float32),
            pltpu.VMEM((2, _TCH, 8, 128), jnp.float32),
            pltpu.VMEM((C * 104,), jnp.int32),
            pltpu.VMEM((2 * C * 16,), jnp.float32),
            pltpu.SemaphoreType.DMA,
            pltpu.SemaphoreType.DMA,
            pltpu.SemaphoreType.DMA,
            pltpu.SemaphoreType.DMA,
        ],
    )
    def _run(x_hbm, pF_hbm, ab_hbm, out_hbm,
             xbuf, obuf, ptab, abv,
             sem_in0, sem_in1, sem_out0, sem_out1):
        wid = lax.axis_index("s") * _NC + lax.axis_index("c")
        grp = wid & 1
        base = (wid >> 1) * tiles_w

        pltpu.sync_copy(pF_hbm, ptab)
        pltpu.sync_copy(ab_hbm, abv)

        avs = [abv[pl.ds((grp * 8 + i) * 16, 16)] for i in range(8)]
        bvs = [abv[pl.ds((C + grp * 8 + i) * 16, 16)] for i in range(8)]
        col0 = grp * 8
        sems_in = (sem_in0, sem_in1)
        sems_out = (sem_out0, sem_out1)

        def start_in(slot, k):
            t0 = base + k * _TCH
            pltpu.async_copy(x_hbm.at[grp, pl.ds(t0, _TCH)],
                             xbuf.at[slot], sems_in[slot])

        def wait_in(slot):
            pltpu.make_async_copy(x_hbm.at[0, pl.ds(0, _TCH)],
                                  xbuf.at[slot], sems_in[slot]).wait()

        def start_out(slot, k):
            t0 = base + k * _TCH
            pltpu.async_copy(obuf.at[slot],
                             out_hbm.at[grp, pl.ds(t0, _TCH)],
                             sems_out[slot])

        def wait_out(slot):
            pltpu.make_async_copy(obuf.at[slot],
                                  out_hbm.at[0, pl.ds(0, _TCH)],
                                  sems_out[slot]).wait()

        def compute_chunk(slot):
            xv = xbuf.at[slot].reshape(_TCH * 8, 128)
            ov = obuf.at[slot].reshape(_TCH * 8, 128)
            himask = jnp.int32(-65536)
            for i in range(8):
                ptab_i = ptab.at[pl.ds((col0 + i) * _ST, _ST)]

                @plsc.parallel_loop(i, _TCH * 8, 8, unroll=1)
                def _(r):
                    for jj in range(8):
                        x = xv[r, pl.ds(16 * jj, 16)]
                        tt = x * avs[i] + bvs[i]
                        idx = tt.astype(jnp.int32)
                        frac = tt - idx.astype(jnp.float32)
                        w = plsc.load_gather(ptab_i, [idx])
                        ylo = plsc.bitcast(w & himask, jnp.float32)
                        dy = plsc.bitcast(w << 16, jnp.float32)
                        ov[r, pl.ds(16 * jj, 16)] = ylo + frac * dy

        start_in(0, 0)
        start_in(1, 1)

        def pair_step(p, _):
            k0 = 2 * p
            for sub in (0, 1):
                k = k0 + sub
                wait_in(sub)

                @pl.when(k >= 2)
                def _():
                    wait_out(sub)

                compute_chunk(sub)
                start_out(sub, k)

                @pl.when(k + 2 < n_chunks)
                def _():
                    start_in(sub, k + 2)
            return 0

        lax.fori_loop(0, n_chunks // 2, pair_step, 0)
        wait_out(0)
        wait_out(1)

    o4 = _run(x4, pF, ab)
    return jnp.transpose(o4.transpose(0, 2, 1, 3).reshape(C, N))
